# baseline (speedup 1.0000x reference)
import math

import numpy as np
import jax
import jax.numpy as jnp
from jax.experimental import pallas as pl
from jax.experimental.pallas import tpu as pltpu

_NT = 2048
_D = 2048
_NE = 16
_CAP = 256
_EPS = float(jnp.finfo(jnp.float32).eps)

_TB = 256
_NB = _NT // _TB

_GUMBEL = np.asarray(
    jax.random.gumbel(jax.random.key(42), (_NT, _NE), dtype=jnp.float32))


def _cumsum_tokens(m):
    s = 1
    while s < _NT:
        shifted = jnp.pad(m[:-s, :], ((s, 0), (0, 0)))
        m = m + shifted
        s *= 2
    return m


def _first_argmax(vals, e_iota):
    vmax = jnp.max(vals, axis=1, keepdims=True)
    return jnp.min(jnp.where(vals == vmax, e_iota, _NE), axis=1)


def _route_kernel(x_ref, wg_ref, gum_ref,
                  g1_ref, g2_ref, l1_ref, l2_ref, laux_ref,
                  logits_scr):
    i = pl.program_id(0)
    blk = jnp.dot(x_ref[...], wg_ref[...], preferred_element_type=jnp.float32)
    logits_scr[pl.ds(i * _TB, _TB), :] = blk

    @pl.when(i == _NB - 1)
    def _():
        logits = logits_scr[...]
        lmax = jnp.max(logits, axis=1, keepdims=True)
        unnorm = jnp.exp(logits - lmax)
        gates = unnorm / jnp.sum(unnorm, axis=1, keepdims=True)

        e_iota = jax.lax.broadcasted_iota(jnp.int32, (_NT, _NE), 1)

        i1 = _first_argmax(gates, e_iota)
        m1 = (e_iota == i1[:, None])
        mask1 = m1.astype(jnp.float32)

        noised = logits + gum_ref[...]
        noised = jnp.where(m1, -jnp.inf, noised)
        i2 = _first_argmax(noised, e_iota)
        m2 = (e_iota == i2[:, None])
        mask2 = m2.astype(jnp.float32)

        cs1 = _cumsum_tokens(mask1)
        locations1 = cs1 - 1.0
        count1 = cs1[_NT - 1:_NT, :]
        locations2 = (_cumsum_tokens(mask2) - 1.0) + count1

        me = jnp.mean(gates, axis=0)
        ce = jnp.mean(mask1, axis=0)
        laux_ref[...] = (jnp.mean(me * ce) * (_NE * _NE)).reshape(1, 1)

        keep1 = (locations1 < _CAP).astype(jnp.float32)
        keep2 = (locations2 < _CAP).astype(jnp.float32)
        mask1 = mask1 * keep1
        mask2 = mask2 * keep2

        g1s = jnp.sum(gates * mask1, axis=1)
        g2s = jnp.sum(gates * mask2, axis=1)
        denom = jnp.maximum(g1s + g2s, _EPS)
        g1s = g1s / denom
        g2s = g2s / denom

        l1s = jnp.sum(locations1 * mask1, axis=1).astype(jnp.int32)
        l2s = jnp.sum(locations2 * mask2, axis=1).astype(jnp.int32)

        g1_ref[...] = g1s[:, None] * mask1
        g2_ref[...] = g2s[:, None] * mask2
        l1_ref[...] = jnp.broadcast_to(l1s[:, None], (_NT, _NE))
        l2_ref[...] = jnp.broadcast_to(l2s[:, None], (_NT, _NE))


def _combine_kernel(g1_ref, g2_ref, l1_ref, l2_ref, cw_ref, dm_ref):
    g1 = g1_ref[...]
    g2 = g2_ref[...]
    l1 = l1_ref[...]
    l2 = l2_ref[...]
    c_iota = jax.lax.broadcasted_iota(jnp.int32, (_TB, _NE, _CAP), 2)
    cw = (g1[:, :, None] * (c_iota == l1[:, :, None]).astype(jnp.float32)
          + g2[:, :, None] * (c_iota == l2[:, :, None]).astype(jnp.float32))
    cw_ref[...] = cw
    dm_ref[...] = cw > 0.0


def kernel(x, Wg):
    gum = jnp.asarray(_GUMBEL)

    small = pl.BlockSpec((_NT, _NE), lambda i: (0, 0))
    g1, g2, l1, l2, laux = pl.pallas_call(
        _route_kernel,
        grid=(_NB,),
        in_specs=[
            pl.BlockSpec((_TB, _D), lambda i: (i, 0)),
            pl.BlockSpec((_D, _NE), lambda i: (0, 0)),
            pl.BlockSpec((_NT, _NE), lambda i: (0, 0)),
        ],
        out_specs=[small, small, small, small,
                   pl.BlockSpec((1, 1), lambda i: (0, 0))],
        out_shape=[
            jax.ShapeDtypeStruct((_NT, _NE), jnp.float32),
            jax.ShapeDtypeStruct((_NT, _NE), jnp.float32),
            jax.ShapeDtypeStruct((_NT, _NE), jnp.int32),
            jax.ShapeDtypeStruct((_NT, _NE), jnp.int32),
            jax.ShapeDtypeStruct((1, 1), jnp.float32),
        ],
        scratch_shapes=[pltpu.VMEM((_NT, _NE), jnp.float32)],
    )(x, Wg, gum)

    tok = pl.BlockSpec((_TB, _NE), lambda i: (i, 0))
    big = pl.BlockSpec((_TB, _NE, _CAP), lambda i: (i, 0, 0))
    cw, dm = pl.pallas_call(
        _combine_kernel,
        grid=(_NB,),
        in_specs=[tok, tok, tok, tok],
        out_specs=[big, big],
        out_shape=[
            jax.ShapeDtypeStruct((_NT, _NE, _CAP), jnp.float32),
            jax.ShapeDtypeStruct((_NT, _NE, _CAP), jnp.bool_),
        ],
    )(g1, g2, l1, l2)

    return laux[0, 0], cw, dm
